# Initial kernel scaffold; baseline (speedup 1.0000x reference)
import jax
import jax.numpy as jnp
from jax.experimental import pallas as pl


def kernel(edge_index, h, e, params):
    raise NotImplementedError("write your pallas kernel here")


# speedup vs baseline: 4.5093x; 4.5093x over previous
import jax
import jax.numpy as jnp
from jax import lax
from jax.experimental import pallas as pl
from jax.experimental.pallas import tpu as pltpu
from jax.experimental.pallas import tpu_sc as plsc

N = 10000
E = 320000
HID = 128
HALF = HID // 2
NCLS = 8
ASSIGN = 100
SIGMA = 0.5
EPS = 1e-5

NC = 2
NS = 16
BATCH = 128
NB = 2560
E_PAD = NB * BATCH
NBT = NB // NS
G = 8
NGROUPS = NBT // G
N_PAD = 10240
ROWS_PER_TILE = N_PAD // NS
DEGW = 16

_sc_mesh = plsc.VectorSubcoreMesh(core_axis_name="c", subcore_axis_name="s")


def _agg_body(compute_deg, h_hbm, src2, dst2, z_hbm, zdeg_hbm, ones_hbm,
              out_hbm, deg_hbm, src_v, dst_v, rows_v, ones_v, acc_sh,
              deg_sh, gsem, ssem):
    c = lax.axis_index("c")
    s = lax.axis_index("s")
    base = s * NBT

    pltpu.sync_copy(z_hbm, acc_sh.at[pl.ds(s * ROWS_PER_TILE, ROWS_PER_TILE)])
    if compute_deg:
        @pl.when(c == 0)
        def _():
            pltpu.sync_copy(
                zdeg_hbm, deg_sh.at[pl.ds(s * ROWS_PER_TILE, ROWS_PER_TILE)])
            pltpu.sync_copy(ones_hbm, ones_v)
    plsc.subcore_barrier()

    def drain_scatters():
        for i in range(G):
            pltpu.make_async_copy(h_hbm.at[pl.ds(0, BATCH)],
                                  rows_v.at[i], ssem).wait()

    def group(g, carry):
        p = g % 2
        b0 = base + g * G
        pltpu.sync_copy(src2.at[c, pl.ds(b0, G)], src_v.at[p])
        pltpu.sync_copy(dst2.at[pl.ds(b0, G)], dst_v.at[p])

        @pl.when(g > 0)
        def _():
            drain_scatters()

        descs = [pltpu.async_copy(h_hbm.at[src_v.at[p, i]],
                                  rows_v.at[i], gsem)
                 for i in range(G)]
        for i in range(G):
            descs[i].wait()
            pltpu.async_copy(rows_v.at[i], acc_sh.at[dst_v.at[p, i]], ssem,
                             add=True)
            if compute_deg:
                @pl.when(c == 0)
                def _():
                    pltpu.sync_copy(
                        ones_v, deg_sh.at[dst_v.at[p, i]], add=True)
        return carry

    lax.fori_loop(0, NGROUPS, group, 0)
    drain_scatters()
    plsc.subcore_barrier()
    r0 = s * ROWS_PER_TILE
    pltpu.sync_copy(acc_sh.at[pl.ds(r0, ROWS_PER_TILE)],
                    out_hbm.at[c, pl.ds(r0, ROWS_PER_TILE)])
    if compute_deg:
        @pl.when(c == 0)
        def _():
            pltpu.sync_copy(deg_sh.at[pl.ds(r0, ROWS_PER_TILE)],
                            deg_hbm.at[pl.ds(r0, ROWS_PER_TILE)])


def _make_sc_agg(compute_deg):
    out_type = [jax.ShapeDtypeStruct((NC, N_PAD, HALF), jnp.float32)]
    if compute_deg:
        out_type.append(jax.ShapeDtypeStruct((N_PAD, DEGW), jnp.float32))
    scratch = [
        pltpu.VMEM((2, G, BATCH), jnp.int32),
        pltpu.VMEM((2, G, BATCH), jnp.int32),
        pltpu.VMEM((G, BATCH, HALF), jnp.float32),
        pltpu.VMEM((BATCH, DEGW), jnp.float32),
        pltpu.VMEM_SHARED((N_PAD, HALF), jnp.float32),
        pltpu.VMEM_SHARED((N_PAD, DEGW), jnp.float32),
        pltpu.SemaphoreType.DMA,
        pltpu.SemaphoreType.DMA,
    ]

    if compute_deg:
        def body(h_hbm, src2, dst2, z_hbm, zdeg_hbm, ones_hbm, out_hbm,
                 deg_hbm, *scr):
            _agg_body(True, h_hbm, src2, dst2, z_hbm, zdeg_hbm, ones_hbm,
                      out_hbm, deg_hbm, *scr)
    else:
        def body(h_hbm, src2, dst2, z_hbm, zdeg_hbm, ones_hbm, out_hbm,
                 *scr):
            _agg_body(False, h_hbm, src2, dst2, z_hbm, zdeg_hbm, ones_hbm,
                      out_hbm, None, *scr)

    return pl.kernel(body, out_type=out_type, mesh=_sc_mesh,
                     scratch_types=scratch,
                     compiler_params=pltpu.CompilerParams(
                         use_tc_tiling_on_sc=False))


_sc_agg_deg = _make_sc_agg(True)
_sc_agg = _make_sc_agg(False)


def _split(h):
    return jnp.concatenate([h[:, :HALF], h[:, HALF:]], axis=0)


def _embed_body(hidx_ref, emb_ref, out_ref, split_ref):
    idx = hidx_ref[...]
    cols = lax.broadcasted_iota(jnp.int32, (1, HID), 1)
    oh = (idx == cols).astype(jnp.float32)
    h0 = lax.dot_general(oh, emb_ref[...], (((1,), (0,)), ((), ())),
                         preferred_element_type=jnp.float32)
    out_ref[...] = h0
    split_ref[...] = _split(h0)


def _bn_relu_res(h2, hin, g, be, rm, rv):
    h2 = (h2 - rm) * lax.rsqrt(rv + EPS) * g + be
    return hin + jnp.maximum(h2, 0.0)


def _stitch(part_ref):
    return jnp.concatenate([part_ref[0, :N], part_ref[1, :N]], axis=1)


def _layer0_body(part_ref, degp_ref, hin_ref, w_ref, b_ref, g_ref, be_ref,
                 rm_ref, rv_ref, hout_ref, split_ref, invdeg_ref):
    deg = degp_ref[:N, 0:1]
    invdeg = 1.0 / jnp.maximum(deg, 1.0)
    invdeg_ref[...] = invdeg
    agg = _stitch(part_ref) * invdeg
    h2 = lax.dot_general(agg, w_ref[...], (((1,), (0,)), ((), ())),
                         preferred_element_type=jnp.float32) + b_ref[...]
    hout = _bn_relu_res(h2, hin_ref[...], g_ref[...], be_ref[...],
                        rm_ref[...], rv_ref[...])
    hout_ref[...] = hout
    split_ref[...] = _split(hout)


def _layer2_body(part_ref, invdeg_ref, hin_ref, w_ref, b_ref, g_ref, be_ref,
                 rm_ref, rv_ref, hout_ref, split_ref):
    agg = _stitch(part_ref) * invdeg_ref[...]
    h2 = lax.dot_general(agg, w_ref[...], (((1,), (0,)), ((), ())),
                         preferred_element_type=jnp.float32) + b_ref[...]
    hout = _bn_relu_res(h2, hin_ref[...], g_ref[...], be_ref[...],
                        rm_ref[...], rv_ref[...])
    hout_ref[...] = hout
    split_ref[...] = _split(hout)


def _layer1_body(part_ref, invdeg_ref, hin_ref, w_ref, b_ref, wa_ref, ba_ref,
                 g_ref, be_ref, rm_ref, rv_ref, hout_ref, split_ref, s_ref):
    agg = _stitch(part_ref) * invdeg_ref[...]
    h2 = lax.dot_general(agg, w_ref[...], (((1,), (0,)), ((), ())),
                         preferred_element_type=jnp.float32) + b_ref[...]
    logits = lax.dot_general(agg, wa_ref[...], (((1,), (0,)), ((), ())),
                             preferred_element_type=jnp.float32) + ba_ref[...]
    m = jnp.max(logits, axis=-1, keepdims=True)
    ex = jnp.exp(logits - m)
    sm = ex / jnp.sum(ex, axis=-1, keepdims=True)
    pooled = lax.dot_general(sm, h2, (((0,), (0,)), ((), ())),
                             preferred_element_type=jnp.float32)
    unpool = lax.dot_general(sm, pooled, (((1,), (0,)), ((), ())),
                             preferred_element_type=jnp.float32)
    h2 = SIGMA * h2 + (1.0 - SIGMA) * unpool
    s_ref[...] = sm
    hout = _bn_relu_res(h2, hin_ref[...], g_ref[...], be_ref[...],
                        rm_ref[...], rv_ref[...])
    hout_ref[...] = hout
    split_ref[...] = _split(hout)


def _layer3_head_body(part_ref, invdeg_ref, hin_ref, w_ref, b_ref, g_ref,
                      be_ref, rm_ref, rv_ref, m0_ref, mb0_ref, m1_ref,
                      mb1_ref, m2_ref, mb2_ref, hout_ref, logits_ref):
    agg = _stitch(part_ref) * invdeg_ref[...]
    h2 = lax.dot_general(agg, w_ref[...], (((1,), (0,)), ((), ())),
                         preferred_element_type=jnp.float32) + b_ref[...]
    h4 = _bn_relu_res(h2, hin_ref[...], g_ref[...], be_ref[...],
                      rm_ref[...], rv_ref[...])
    hout_ref[...] = h4
    y = jnp.maximum(lax.dot_general(h4, m0_ref[...], (((1,), (0,)), ((), ())),
                                    preferred_element_type=jnp.float32)
                    + mb0_ref[...], 0.0)
    y = jnp.maximum(lax.dot_general(y, m1_ref[...], (((1,), (0,)), ((), ())),
                                    preferred_element_type=jnp.float32)
                    + mb1_ref[...], 0.0)
    logits_ref[...] = lax.dot_general(y, m2_ref[...], (((1,), (0,)), ((), ())),
                                      preferred_element_type=jnp.float32) \
        + mb2_ref[...]


def _f32(shape):
    return jax.ShapeDtypeStruct(shape, jnp.float32)


_SPLIT_T = _f32((2 * N, HALF))

_embed = pl.pallas_call(_embed_body, out_shape=(_f32((N, HID)), _SPLIT_T))
_layer0 = pl.pallas_call(
    _layer0_body, out_shape=(_f32((N, HID)), _SPLIT_T, _f32((N, 1))))
_layer2 = pl.pallas_call(_layer2_body, out_shape=(_f32((N, HID)), _SPLIT_T))
_layer1 = pl.pallas_call(
    _layer1_body, out_shape=(_f32((N, HID)), _SPLIT_T, _f32((N, ASSIGN))))
_layer3 = pl.pallas_call(_layer3_head_body,
                         out_shape=(_f32((N, HID)), _f32((N, NCLS))))


def kernel(edge_index, h, e, params):
    src = edge_index[0]
    dst = edge_index[1]
    pad = E_PAD - E
    src_p = jnp.concatenate([src, jnp.zeros((pad,), jnp.int32)])
    src2 = jnp.stack([src_p, src_p + N]).reshape(2, NB, BATCH)
    dst2 = jnp.concatenate(
        [dst, jnp.full((pad,), N, jnp.int32)]).reshape(NB, BATCH)
    z = jnp.zeros((ROWS_PER_TILE, HALF), jnp.float32)
    zdeg = jnp.zeros((ROWS_PER_TILE, DEGW), jnp.float32)
    ones = jnp.ones((BATCH, DEGW), jnp.float32)
    p = params

    h0, hs0 = _embed(h.reshape(N, 1), p['emb'])

    def row(v):
        return v.reshape(1, -1)

    part0, degp = _sc_agg_deg(hs0, src2, dst2, z, zdeg, ones)
    h1, hs1, invdeg = _layer0(part0, degp, h0, p['W0'], row(p['b0']),
                              row(p['g0']), row(p['be0']), row(p['rm0']),
                              row(p['rv0']))

    part1, = _sc_agg(hs1, src2, dst2, z, zdeg, ones)
    h2, hs2, s_out = _layer1(part1, invdeg, h1, p['W1'], row(p['b1']),
                             p['Wa'], row(p['ba']), row(p['g1']),
                             row(p['be1']), row(p['rm1']), row(p['rv1']))

    part2, = _sc_agg(hs2, src2, dst2, z, zdeg, ones)
    h3, hs3 = _layer2(part2, invdeg, h2, p['W2'], row(p['b2']), row(p['g2']),
                      row(p['be2']), row(p['rm2']), row(p['rv2']))

    part3, = _sc_agg(hs3, src2, dst2, z, zdeg, ones)
    h4, h_out = _layer3(part3, invdeg, h3, p['W3'], row(p['b3']), row(p['g3']),
                        row(p['be3']), row(p['rm3']), row(p['rv3']),
                        p['M0'], row(p['mb0']), p['M1'], row(p['mb1']),
                        p['M2'], row(p['mb2']))

    Hs = jnp.stack([h0, h1, h2, h3, h4], axis=0)
    return (h_out, Hs, s_out)


# speedup vs baseline: 4.6043x; 1.0211x over previous
import jax
import jax.numpy as jnp
from jax import lax
from jax.experimental import pallas as pl
from jax.experimental.pallas import tpu as pltpu
from jax.experimental.pallas import tpu_sc as plsc

N = 10000
E = 320000
HID = 128
HALF = HID // 2
NCLS = 8
ASSIGN = 100
SIGMA = 0.5
EPS = 1e-5

NC = 2
NS = 16
BATCH = 128
NB = 2560
E_PAD = NB * BATCH
NBT = NB // NS
G = 4
NGROUPS = NBT // G
N_PAD = 10240
ROWS_PER_TILE = N_PAD // NS
DEGW = 16

_sc_mesh = plsc.VectorSubcoreMesh(core_axis_name="c", subcore_axis_name="s")


def _agg_body(compute_deg, h_hbm, src2, dst2, z_hbm, zdeg_hbm, ones_hbm,
              out_hbm, deg_hbm, src_v, dst_v, rows_v, ones_v, acc_sh,
              deg_sh, gsem, ssem):
    c = lax.axis_index("c")
    s = lax.axis_index("s")
    base = s * NBT

    pltpu.sync_copy(z_hbm, acc_sh.at[pl.ds(s * ROWS_PER_TILE, ROWS_PER_TILE)])
    if compute_deg:
        @pl.when(c == 0)
        def _():
            pltpu.sync_copy(
                zdeg_hbm, deg_sh.at[pl.ds(s * ROWS_PER_TILE, ROWS_PER_TILE)])
            pltpu.sync_copy(ones_hbm, ones_v)
    plsc.subcore_barrier()

    def load_and_gather(g, p):
        b0 = base + g * G
        pltpu.sync_copy(src2.at[c, pl.ds(b0, G)], src_v.at[p])
        pltpu.sync_copy(dst2.at[pl.ds(b0, G)], dst_v.at[p])
        for i in range(G):
            pltpu.async_copy(h_hbm.at[src_v.at[p, i]], rows_v.at[p, i], gsem)

    def drain_scatters():
        for i in range(G):
            pltpu.make_async_copy(h_hbm.at[pl.ds(0, BATCH)],
                                  rows_v.at[0, i], ssem).wait()

    load_and_gather(0, 0)

    def group(g, carry):
        p = g % 2
        @pl.when(g + 1 < NGROUPS)
        def _():
            @pl.when(g > 0)
            def _():
                drain_scatters()
            load_and_gather(g + 1, 1 - p)

        for i in range(G):
            pltpu.make_async_copy(h_hbm.at[pl.ds(0, BATCH)],
                                  rows_v.at[p, i], gsem).wait()
            pltpu.async_copy(rows_v.at[p, i], acc_sh.at[dst_v.at[p, i]],
                             ssem, add=True)
            if compute_deg:
                @pl.when(c == 0)
                def _():
                    pltpu.sync_copy(
                        ones_v, deg_sh.at[dst_v.at[p, i]], add=True)
        return carry

    lax.fori_loop(0, NGROUPS, group, 0)
    drain_scatters()
    drain_scatters()
    plsc.subcore_barrier()
    r0 = s * ROWS_PER_TILE
    pltpu.sync_copy(acc_sh.at[pl.ds(r0, ROWS_PER_TILE)],
                    out_hbm.at[c, pl.ds(r0, ROWS_PER_TILE)])
    if compute_deg:
        @pl.when(c == 0)
        def _():
            pltpu.sync_copy(deg_sh.at[pl.ds(r0, ROWS_PER_TILE)],
                            deg_hbm.at[pl.ds(r0, ROWS_PER_TILE)])


def _make_sc_agg(compute_deg):
    out_type = [jax.ShapeDtypeStruct((NC, N_PAD, HALF), jnp.float32)]
    if compute_deg:
        out_type.append(jax.ShapeDtypeStruct((N_PAD, DEGW), jnp.float32))
    scratch = [
        pltpu.VMEM((2, G, BATCH), jnp.int32),
        pltpu.VMEM((2, G, BATCH), jnp.int32),
        pltpu.VMEM((2, G, BATCH, HALF), jnp.float32),
        pltpu.VMEM((BATCH, DEGW), jnp.float32),
        pltpu.VMEM_SHARED((N_PAD, HALF), jnp.float32),
        pltpu.VMEM_SHARED((N_PAD, DEGW), jnp.float32),
        pltpu.SemaphoreType.DMA,
        pltpu.SemaphoreType.DMA,
    ]

    if compute_deg:
        def body(h_hbm, src2, dst2, z_hbm, zdeg_hbm, ones_hbm, out_hbm,
                 deg_hbm, *scr):
            _agg_body(True, h_hbm, src2, dst2, z_hbm, zdeg_hbm, ones_hbm,
                      out_hbm, deg_hbm, *scr)
    else:
        def body(h_hbm, src2, dst2, z_hbm, zdeg_hbm, ones_hbm, out_hbm,
                 *scr):
            _agg_body(False, h_hbm, src2, dst2, z_hbm, zdeg_hbm, ones_hbm,
                      out_hbm, None, *scr)

    return pl.kernel(body, out_type=out_type, mesh=_sc_mesh,
                     scratch_types=scratch,
                     compiler_params=pltpu.CompilerParams(
                         use_tc_tiling_on_sc=False))


_sc_agg_deg = _make_sc_agg(True)
_sc_agg = _make_sc_agg(False)


def _split(h):
    return jnp.concatenate([h[:, :HALF], h[:, HALF:]], axis=0)


def _embed_body(hidx_ref, emb_ref, out_ref, split_ref):
    idx = hidx_ref[...]
    cols = lax.broadcasted_iota(jnp.int32, (1, HID), 1)
    oh = (idx == cols).astype(jnp.float32)
    h0 = lax.dot_general(oh, emb_ref[...], (((1,), (0,)), ((), ())),
                         preferred_element_type=jnp.float32)
    out_ref[...] = h0
    split_ref[...] = _split(h0)


def _bn_relu_res(h2, hin, g, be, rm, rv):
    h2 = (h2 - rm) * lax.rsqrt(rv + EPS) * g + be
    return hin + jnp.maximum(h2, 0.0)


def _stitch(part_ref):
    return jnp.concatenate([part_ref[0, :N], part_ref[1, :N]], axis=1)


def _layer0_body(part_ref, degp_ref, hin_ref, w_ref, b_ref, g_ref, be_ref,
                 rm_ref, rv_ref, hout_ref, split_ref, invdeg_ref):
    deg = degp_ref[:N, 0:1]
    invdeg = 1.0 / jnp.maximum(deg, 1.0)
    invdeg_ref[...] = invdeg
    agg = _stitch(part_ref) * invdeg
    h2 = lax.dot_general(agg, w_ref[...], (((1,), (0,)), ((), ())),
                         preferred_element_type=jnp.float32) + b_ref[...]
    hout = _bn_relu_res(h2, hin_ref[...], g_ref[...], be_ref[...],
                        rm_ref[...], rv_ref[...])
    hout_ref[...] = hout
    split_ref[...] = _split(hout)


def _layer2_body(part_ref, invdeg_ref, hin_ref, w_ref, b_ref, g_ref, be_ref,
                 rm_ref, rv_ref, hout_ref, split_ref):
    agg = _stitch(part_ref) * invdeg_ref[...]
    h2 = lax.dot_general(agg, w_ref[...], (((1,), (0,)), ((), ())),
                         preferred_element_type=jnp.float32) + b_ref[...]
    hout = _bn_relu_res(h2, hin_ref[...], g_ref[...], be_ref[...],
                        rm_ref[...], rv_ref[...])
    hout_ref[...] = hout
    split_ref[...] = _split(hout)


def _layer1_body(part_ref, invdeg_ref, hin_ref, w_ref, b_ref, wa_ref, ba_ref,
                 g_ref, be_ref, rm_ref, rv_ref, hout_ref, split_ref, s_ref):
    agg = _stitch(part_ref) * invdeg_ref[...]
    h2 = lax.dot_general(agg, w_ref[...], (((1,), (0,)), ((), ())),
                         preferred_element_type=jnp.float32) + b_ref[...]
    logits = lax.dot_general(agg, wa_ref[...], (((1,), (0,)), ((), ())),
                             preferred_element_type=jnp.float32) + ba_ref[...]
    m = jnp.max(logits, axis=-1, keepdims=True)
    ex = jnp.exp(logits - m)
    sm = ex / jnp.sum(ex, axis=-1, keepdims=True)
    pooled = lax.dot_general(sm, h2, (((0,), (0,)), ((), ())),
                             preferred_element_type=jnp.float32)
    unpool = lax.dot_general(sm, pooled, (((1,), (0,)), ((), ())),
                             preferred_element_type=jnp.float32)
    h2 = SIGMA * h2 + (1.0 - SIGMA) * unpool
    s_ref[...] = sm
    hout = _bn_relu_res(h2, hin_ref[...], g_ref[...], be_ref[...],
                        rm_ref[...], rv_ref[...])
    hout_ref[...] = hout
    split_ref[...] = _split(hout)


def _layer3_head_body(part_ref, invdeg_ref, hin_ref, w_ref, b_ref, g_ref,
                      be_ref, rm_ref, rv_ref, m0_ref, mb0_ref, m1_ref,
                      mb1_ref, m2_ref, mb2_ref, hout_ref, logits_ref):
    agg = _stitch(part_ref) * invdeg_ref[...]
    h2 = lax.dot_general(agg, w_ref[...], (((1,), (0,)), ((), ())),
                         preferred_element_type=jnp.float32) + b_ref[...]
    h4 = _bn_relu_res(h2, hin_ref[...], g_ref[...], be_ref[...],
                      rm_ref[...], rv_ref[...])
    hout_ref[...] = h4
    y = jnp.maximum(lax.dot_general(h4, m0_ref[...], (((1,), (0,)), ((), ())),
                                    preferred_element_type=jnp.float32)
                    + mb0_ref[...], 0.0)
    y = jnp.maximum(lax.dot_general(y, m1_ref[...], (((1,), (0,)), ((), ())),
                                    preferred_element_type=jnp.float32)
                    + mb1_ref[...], 0.0)
    logits_ref[...] = lax.dot_general(y, m2_ref[...], (((1,), (0,)), ((), ())),
                                      preferred_element_type=jnp.float32) \
        + mb2_ref[...]


def _f32(shape):
    return jax.ShapeDtypeStruct(shape, jnp.float32)


_SPLIT_T = _f32((2 * N, HALF))

_embed = pl.pallas_call(_embed_body, out_shape=(_f32((N, HID)), _SPLIT_T))
_layer0 = pl.pallas_call(
    _layer0_body, out_shape=(_f32((N, HID)), _SPLIT_T, _f32((N, 1))))
_layer2 = pl.pallas_call(_layer2_body, out_shape=(_f32((N, HID)), _SPLIT_T))
_layer1 = pl.pallas_call(
    _layer1_body, out_shape=(_f32((N, HID)), _SPLIT_T, _f32((N, ASSIGN))))
_layer3 = pl.pallas_call(_layer3_head_body,
                         out_shape=(_f32((N, HID)), _f32((N, NCLS))))


def kernel(edge_index, h, e, params):
    src = edge_index[0]
    dst = edge_index[1]
    pad = E_PAD - E
    src_p = jnp.concatenate([src, jnp.zeros((pad,), jnp.int32)])
    src2 = jnp.stack([src_p, src_p + N]).reshape(2, NB, BATCH)
    dst2 = jnp.concatenate(
        [dst, jnp.full((pad,), N, jnp.int32)]).reshape(NB, BATCH)
    z = jnp.zeros((ROWS_PER_TILE, HALF), jnp.float32)
    zdeg = jnp.zeros((ROWS_PER_TILE, DEGW), jnp.float32)
    ones = jnp.ones((BATCH, DEGW), jnp.float32)
    p = params

    h0, hs0 = _embed(h.reshape(N, 1), p['emb'])

    def row(v):
        return v.reshape(1, -1)

    part0, degp = _sc_agg_deg(hs0, src2, dst2, z, zdeg, ones)
    h1, hs1, invdeg = _layer0(part0, degp, h0, p['W0'], row(p['b0']),
                              row(p['g0']), row(p['be0']), row(p['rm0']),
                              row(p['rv0']))

    part1, = _sc_agg(hs1, src2, dst2, z, zdeg, ones)
    h2, hs2, s_out = _layer1(part1, invdeg, h1, p['W1'], row(p['b1']),
                             p['Wa'], row(p['ba']), row(p['g1']),
                             row(p['be1']), row(p['rm1']), row(p['rv1']))

    part2, = _sc_agg(hs2, src2, dst2, z, zdeg, ones)
    h3, hs3 = _layer2(part2, invdeg, h2, p['W2'], row(p['b2']), row(p['g2']),
                      row(p['be2']), row(p['rm2']), row(p['rv2']))

    part3, = _sc_agg(hs3, src2, dst2, z, zdeg, ones)
    h4, h_out = _layer3(part3, invdeg, h3, p['W3'], row(p['b3']), row(p['g3']),
                        row(p['be3']), row(p['rm3']), row(p['rv3']),
                        p['M0'], row(p['mb0']), p['M1'], row(p['mb1']),
                        p['M2'], row(p['mb2']))

    Hs = jnp.stack([h0, h1, h2, h3, h4], axis=0)
    return (h_out, Hs, s_out)


# speedup vs baseline: 4.9036x; 1.0650x over previous
import jax
import jax.numpy as jnp
from jax import lax
from jax.experimental import pallas as pl
from jax.experimental.pallas import tpu as pltpu
from jax.experimental.pallas import tpu_sc as plsc

N = 10000
E = 320000
HID = 128
HALF = HID // 2
NCLS = 8
ASSIGN = 100
SIGMA = 0.5
EPS = 1e-5

NC = 2
NS = 16
BATCH = 128
NB = 2560
E_PAD = NB * BATCH
NBT = NB // NS
G = 4
NGROUPS = NBT // G
N_PAD = 10240
ROWS_PER_TILE = N_PAD // NS
DEGW = 16

_sc_mesh = plsc.VectorSubcoreMesh(core_axis_name="c", subcore_axis_name="s")


def _agg_body(compute_deg, h_hbm, src2, dst2, z_hbm, zdeg_hbm, ones_hbm,
              out_hbm, deg_hbm, src_v, dst_v, rows_v, ones_v, acc_sh,
              deg_sh, gsem, ssem, isem):
    c = lax.axis_index("c")
    s = lax.axis_index("s")
    base = s * NBT

    def load_idx(g, q):
        b0 = base + g * G
        pltpu.async_copy(src2.at[c, pl.ds(b0, G)], src_v.at[q], isem)
        pltpu.async_copy(dst2.at[pl.ds(b0, G)], dst_v.at[q], isem)

    def wait_idx(q):
        pltpu.make_async_copy(src2.at[c, pl.ds(0, G)], src_v.at[q],
                              isem).wait()
        pltpu.make_async_copy(dst2.at[pl.ds(0, G)], dst_v.at[q], isem).wait()

    def issue_gathers(p, q):
        for i in range(G):
            pltpu.async_copy(h_hbm.at[src_v.at[q, i]], rows_v.at[p, i], gsem)

    def drain_scatters():
        for i in range(G):
            pltpu.make_async_copy(h_hbm.at[pl.ds(0, BATCH)],
                                  rows_v.at[0, i], ssem).wait()

    load_idx(0, 0)
    pltpu.sync_copy(z_hbm, acc_sh.at[pl.ds(s * ROWS_PER_TILE, ROWS_PER_TILE)])
    if compute_deg:
        @pl.when(c == 0)
        def _():
            pltpu.sync_copy(
                zdeg_hbm, deg_sh.at[pl.ds(s * ROWS_PER_TILE, ROWS_PER_TILE)])
            pltpu.sync_copy(ones_hbm, ones_v)
    wait_idx(0)
    issue_gathers(0, 0)
    load_idx(1, 1)
    plsc.subcore_barrier()

    def group(g, carry):
        p = g % 2
        q = g % 3
        @pl.when(g + 1 < NGROUPS)
        def _():
            wait_idx((g + 1) % 3)
            @pl.when(g > 0)
            def _():
                drain_scatters()
            issue_gathers(1 - p, (g + 1) % 3)
        @pl.when(g + 2 < NGROUPS)
        def _():
            load_idx(g + 2, (g + 2) % 3)

        for i in range(G):
            pltpu.make_async_copy(h_hbm.at[pl.ds(0, BATCH)],
                                  rows_v.at[p, i], gsem).wait()
            pltpu.async_copy(rows_v.at[p, i], acc_sh.at[dst_v.at[q, i]],
                             ssem, add=True)
            if compute_deg:
                @pl.when(c == 0)
                def _():
                    pltpu.sync_copy(
                        ones_v, deg_sh.at[dst_v.at[q, i]], add=True)
        return carry

    lax.fori_loop(0, NGROUPS, group, 0)
    drain_scatters()
    drain_scatters()
    plsc.subcore_barrier()
    r0 = s * ROWS_PER_TILE
    pltpu.sync_copy(acc_sh.at[pl.ds(r0, ROWS_PER_TILE)],
                    out_hbm.at[c, pl.ds(r0, ROWS_PER_TILE)])
    if compute_deg:
        @pl.when(c == 0)
        def _():
            pltpu.sync_copy(deg_sh.at[pl.ds(r0, ROWS_PER_TILE)],
                            deg_hbm.at[pl.ds(r0, ROWS_PER_TILE)])


def _make_sc_agg(compute_deg):
    out_type = [jax.ShapeDtypeStruct((NC, N_PAD, HALF), jnp.float32)]
    if compute_deg:
        out_type.append(jax.ShapeDtypeStruct((N_PAD, DEGW), jnp.float32))
    scratch = [
        pltpu.VMEM((3, G, BATCH), jnp.int32),
        pltpu.VMEM((3, G, BATCH), jnp.int32),
        pltpu.VMEM((2, G, BATCH, HALF), jnp.float32),
        pltpu.VMEM((BATCH, DEGW), jnp.float32),
        pltpu.VMEM_SHARED((N_PAD, HALF), jnp.float32),
        pltpu.VMEM_SHARED((N_PAD, DEGW), jnp.float32),
        pltpu.SemaphoreType.DMA,
        pltpu.SemaphoreType.DMA,
        pltpu.SemaphoreType.DMA,
    ]

    if compute_deg:
        def body(h_hbm, src2, dst2, z_hbm, zdeg_hbm, ones_hbm, out_hbm,
                 deg_hbm, *scr):
            _agg_body(True, h_hbm, src2, dst2, z_hbm, zdeg_hbm, ones_hbm,
                      out_hbm, deg_hbm, *scr)
    else:
        def body(h_hbm, src2, dst2, z_hbm, zdeg_hbm, ones_hbm, out_hbm,
                 *scr):
            _agg_body(False, h_hbm, src2, dst2, z_hbm, zdeg_hbm, ones_hbm,
                      out_hbm, None, *scr)

    return pl.kernel(body, out_type=out_type, mesh=_sc_mesh,
                     scratch_types=scratch,
                     compiler_params=pltpu.CompilerParams(
                         use_tc_tiling_on_sc=False))


_sc_agg_deg = _make_sc_agg(True)
_sc_agg = _make_sc_agg(False)


def _split(h):
    return jnp.concatenate([h[:, :HALF], h[:, HALF:]], axis=0)


def _embed_body(hidx_ref, emb_ref, out_ref, split_ref):
    idx = hidx_ref[...]
    cols = lax.broadcasted_iota(jnp.int32, (1, HID), 1)
    oh = (idx == cols).astype(jnp.float32)
    h0 = lax.dot_general(oh, emb_ref[...], (((1,), (0,)), ((), ())),
                         preferred_element_type=jnp.float32)
    out_ref[...] = h0
    split_ref[...] = _split(h0)


def _bn_relu_res(h2, hin, g, be, rm, rv):
    h2 = (h2 - rm) * lax.rsqrt(rv + EPS) * g + be
    return hin + jnp.maximum(h2, 0.0)


def _stitch(part_ref):
    return jnp.concatenate([part_ref[0, :N], part_ref[1, :N]], axis=1)


def _layer0_body(part_ref, degp_ref, hin_ref, w_ref, b_ref, g_ref, be_ref,
                 rm_ref, rv_ref, hout_ref, split_ref, invdeg_ref):
    deg = degp_ref[:N, 0:1]
    invdeg = 1.0 / jnp.maximum(deg, 1.0)
    invdeg_ref[...] = invdeg
    agg = _stitch(part_ref) * invdeg
    h2 = lax.dot_general(agg, w_ref[...], (((1,), (0,)), ((), ())),
                         preferred_element_type=jnp.float32) + b_ref[...]
    hout = _bn_relu_res(h2, hin_ref[...], g_ref[...], be_ref[...],
                        rm_ref[...], rv_ref[...])
    hout_ref[...] = hout
    split_ref[...] = _split(hout)


def _layer2_body(part_ref, invdeg_ref, hin_ref, w_ref, b_ref, g_ref, be_ref,
                 rm_ref, rv_ref, hout_ref, split_ref):
    agg = _stitch(part_ref) * invdeg_ref[...]
    h2 = lax.dot_general(agg, w_ref[...], (((1,), (0,)), ((), ())),
                         preferred_element_type=jnp.float32) + b_ref[...]
    hout = _bn_relu_res(h2, hin_ref[...], g_ref[...], be_ref[...],
                        rm_ref[...], rv_ref[...])
    hout_ref[...] = hout
    split_ref[...] = _split(hout)


def _layer1_body(part_ref, invdeg_ref, hin_ref, w_ref, b_ref, wa_ref, ba_ref,
                 g_ref, be_ref, rm_ref, rv_ref, hout_ref, split_ref, s_ref):
    agg = _stitch(part_ref) * invdeg_ref[...]
    h2 = lax.dot_general(agg, w_ref[...], (((1,), (0,)), ((), ())),
                         preferred_element_type=jnp.float32) + b_ref[...]
    logits = lax.dot_general(agg, wa_ref[...], (((1,), (0,)), ((), ())),
                             preferred_element_type=jnp.float32) + ba_ref[...]
    m = jnp.max(logits, axis=-1, keepdims=True)
    ex = jnp.exp(logits - m)
    sm = ex / jnp.sum(ex, axis=-1, keepdims=True)
    pooled = lax.dot_general(sm, h2, (((0,), (0,)), ((), ())),
                             preferred_element_type=jnp.float32)
    unpool = lax.dot_general(sm, pooled, (((1,), (0,)), ((), ())),
                             preferred_element_type=jnp.float32)
    h2 = SIGMA * h2 + (1.0 - SIGMA) * unpool
    s_ref[...] = sm
    hout = _bn_relu_res(h2, hin_ref[...], g_ref[...], be_ref[...],
                        rm_ref[...], rv_ref[...])
    hout_ref[...] = hout
    split_ref[...] = _split(hout)


def _layer3_head_body(part_ref, invdeg_ref, hin_ref, w_ref, b_ref, g_ref,
                      be_ref, rm_ref, rv_ref, m0_ref, mb0_ref, m1_ref,
                      mb1_ref, m2_ref, mb2_ref, hout_ref, logits_ref):
    agg = _stitch(part_ref) * invdeg_ref[...]
    h2 = lax.dot_general(agg, w_ref[...], (((1,), (0,)), ((), ())),
                         preferred_element_type=jnp.float32) + b_ref[...]
    h4 = _bn_relu_res(h2, hin_ref[...], g_ref[...], be_ref[...],
                      rm_ref[...], rv_ref[...])
    hout_ref[...] = h4
    y = jnp.maximum(lax.dot_general(h4, m0_ref[...], (((1,), (0,)), ((), ())),
                                    preferred_element_type=jnp.float32)
                    + mb0_ref[...], 0.0)
    y = jnp.maximum(lax.dot_general(y, m1_ref[...], (((1,), (0,)), ((), ())),
                                    preferred_element_type=jnp.float32)
                    + mb1_ref[...], 0.0)
    logits_ref[...] = lax.dot_general(y, m2_ref[...], (((1,), (0,)), ((), ())),
                                      preferred_element_type=jnp.float32) \
        + mb2_ref[...]


def _f32(shape):
    return jax.ShapeDtypeStruct(shape, jnp.float32)


_SPLIT_T = _f32((2 * N, HALF))

_embed = pl.pallas_call(_embed_body, out_shape=(_f32((N, HID)), _SPLIT_T))
_layer0 = pl.pallas_call(
    _layer0_body, out_shape=(_f32((N, HID)), _SPLIT_T, _f32((N, 1))))
_layer2 = pl.pallas_call(_layer2_body, out_shape=(_f32((N, HID)), _SPLIT_T))
_layer1 = pl.pallas_call(
    _layer1_body, out_shape=(_f32((N, HID)), _SPLIT_T, _f32((N, ASSIGN))))
_layer3 = pl.pallas_call(_layer3_head_body,
                         out_shape=(_f32((N, HID)), _f32((N, NCLS))))


def kernel(edge_index, h, e, params):
    src = edge_index[0]
    dst = edge_index[1]
    pad = E_PAD - E
    src_p = jnp.concatenate([src, jnp.zeros((pad,), jnp.int32)])
    src2 = jnp.stack([src_p, src_p + N]).reshape(2, NB, BATCH)
    dst2 = jnp.concatenate(
        [dst, jnp.full((pad,), N, jnp.int32)]).reshape(NB, BATCH)
    z = jnp.zeros((ROWS_PER_TILE, HALF), jnp.float32)
    zdeg = jnp.zeros((ROWS_PER_TILE, DEGW), jnp.float32)
    ones = jnp.ones((BATCH, DEGW), jnp.float32)
    p = params

    h0, hs0 = _embed(h.reshape(N, 1), p['emb'])

    def row(v):
        return v.reshape(1, -1)

    part0, degp = _sc_agg_deg(hs0, src2, dst2, z, zdeg, ones)
    h1, hs1, invdeg = _layer0(part0, degp, h0, p['W0'], row(p['b0']),
                              row(p['g0']), row(p['be0']), row(p['rm0']),
                              row(p['rv0']))

    part1, = _sc_agg(hs1, src2, dst2, z, zdeg, ones)
    h2, hs2, s_out = _layer1(part1, invdeg, h1, p['W1'], row(p['b1']),
                             p['Wa'], row(p['ba']), row(p['g1']),
                             row(p['be1']), row(p['rm1']), row(p['rv1']))

    part2, = _sc_agg(hs2, src2, dst2, z, zdeg, ones)
    h3, hs3 = _layer2(part2, invdeg, h2, p['W2'], row(p['b2']), row(p['g2']),
                      row(p['be2']), row(p['rm2']), row(p['rv2']))

    part3, = _sc_agg(hs3, src2, dst2, z, zdeg, ones)
    h4, h_out = _layer3(part3, invdeg, h3, p['W3'], row(p['b3']), row(p['g3']),
                        row(p['be3']), row(p['rm3']), row(p['rv3']),
                        p['M0'], row(p['mb0']), p['M1'], row(p['mb1']),
                        p['M2'], row(p['mb2']))

    Hs = jnp.stack([h0, h1, h2, h3, h4], axis=0)
    return (h_out, Hs, s_out)
